# baseline (speedup 1.0000x reference)
import jax
import jax.numpy as jnp
from jax import lax
from jax.experimental import pallas as pl
from jax.experimental.pallas import tpu as pltpu
from jax.experimental.pallas import tpu_sc as plsc

_B = 16384
_D = 64
_N = 1000000
_NC = 2
_NS = 16
_NW = _NC * _NS
_BPW = _B // _NW
_CH = 128
_NCHUNK = _BPW // _CH
_L = 16


def _sc_body(user_id, item_i_id, item_j_id, ut, it,
             pred_i_out, pred_j_out,
             idx_u, idx_i, idx_j, eidx_u, eidx_i, eidx_j,
             rows_u, rows_i, rows_j,
             out_i_v, out_j_v, sem):
    wid = lax.axis_index("s") * _NC + lax.axis_index("c")
    base = wid * _BPW

    ut1 = ut
    it1 = it

    pltpu.sync_copy(user_id.at[pl.ds(base, _BPW)], idx_u)
    pltpu.sync_copy(item_i_id.at[pl.ds(base, _BPW)], idx_i)
    pltpu.sync_copy(item_j_id.at[pl.ds(base, _BPW)], idx_j)

    lanes = lax.iota(jnp.int32, _L)

    def _rot(x, d):
        return x.at[lanes ^ d].get(mode="promise_in_bounds", unique_indices=True)

    def _reduce16(ps):
        vs = ps
        d = _L // 2
        while len(vs) > 1:
            nxt = []
            for m in range(len(vs) // 2):
                a, b = vs[2 * m], vs[2 * m + 1]
                a2 = a + _rot(a, d)
                b2 = b + _rot(b, d)
                nxt.append(jnp.where((lanes & d) == 0, a2, b2))
            vs = nxt
            d //= 2
        br = (((lanes & 1) << 3) | ((lanes & 2) << 1)
              | ((lanes & 4) >> 1) | ((lanes & 8) >> 3))
        return vs[0].at[br].get(mode="promise_in_bounds", unique_indices=True)

    dvecs = [(c * _L + lanes) * _N for c in range(_D // _L)]

    for chunk in range(_NCHUNK):
        c0 = chunk * _CH

        def gen_body(r, carry):
            vu = idx_u[pl.ds(c0 + r, 1)]
            vi = idx_i[pl.ds(c0 + r, 1)]
            vj = idx_j[pl.ds(c0 + r, 1)]
            for c in range(_D // _L):
                sl = pl.ds(r * _D + c * _L, _L)
                eidx_u[sl] = dvecs[c] + vu[0]
                eidx_i[sl] = dvecs[c] + vi[0]
                eidx_j[sl] = dvecs[c] + vj[0]
            return carry

        lax.fori_loop(0, _CH, gen_body, 0)

        cp_u = pltpu.async_copy(ut1.at[eidx_u], rows_u, sem)
        cp_i = pltpu.async_copy(it1.at[eidx_i], rows_i, sem)
        cp_j = pltpu.async_copy(it1.at[eidx_j], rows_j, sem)
        cp_u.wait()
        cp_i.wait()
        cp_j.wait()

        def group_body(g, carry):
            ps_i, ps_j = [], []
            for k in range(_L):
                r = g * _L + k
                p_i = jnp.zeros((_L,), jnp.float32)
                p_j = jnp.zeros((_L,), jnp.float32)
                for c in range(_D // _L):
                    sl = pl.ds(r * _D + c * _L, _L)
                    u = rows_u[sl]
                    p_i = p_i + u * rows_i[sl]
                    p_j = p_j + u * rows_j[sl]
                ps_i.append(p_i)
                ps_j.append(p_j)
            out_i_v[pl.ds(c0 + g * _L, _L)] = _reduce16(ps_i)
            out_j_v[pl.ds(c0 + g * _L, _L)] = _reduce16(ps_j)
            return carry

        lax.fori_loop(0, _CH // _L, group_body, 0)

    pltpu.sync_copy(out_i_v, pred_i_out.at[pl.ds(base, _BPW)])
    pltpu.sync_copy(out_j_v, pred_j_out.at[pl.ds(base, _BPW)])


def kernel(user_id, item_i_id, item_j_id, user_table, item_table):
    ut = user_table.T.reshape(_D * _N)
    it = item_table.T.reshape(_D * _N)
    f = pl.kernel(
        _sc_body,
        mesh=plsc.VectorSubcoreMesh(core_axis_name="c", subcore_axis_name="s"),
        out_type=(
            jax.ShapeDtypeStruct((_B,), jnp.float32),
            jax.ShapeDtypeStruct((_B,), jnp.float32),
        ),
        scratch_types=[
            pltpu.VMEM((_BPW,), jnp.int32),
            pltpu.VMEM((_BPW,), jnp.int32),
            pltpu.VMEM((_BPW,), jnp.int32),
            pltpu.VMEM((_CH * _D,), jnp.int32),
            pltpu.VMEM((_CH * _D,), jnp.int32),
            pltpu.VMEM((_CH * _D,), jnp.int32),
            pltpu.VMEM((_CH * _D,), jnp.float32),
            pltpu.VMEM((_CH * _D,), jnp.float32),
            pltpu.VMEM((_CH * _D,), jnp.float32),
            pltpu.VMEM((_BPW,), jnp.float32),
            pltpu.VMEM((_BPW,), jnp.float32),
            pltpu.SemaphoreType.DMA,
        ],
    )
    return f(user_id, item_i_id, item_j_id, ut, it)


# speedup vs baseline: 14.2635x; 14.2635x over previous
import jax
import jax.numpy as jnp
from jax import lax
from jax.experimental import pallas as pl
from jax.experimental.pallas import tpu as pltpu
from jax.experimental.pallas import tpu_sc as plsc

_B = 16384
_D = 64
_NC = 2
_NS = 16
_NW = _NC * _NS
_BPW = _B // _NW
_CH = 128
_NCHUNK = _BPW // _CH
_L = 16


def _sc_body(user_id, item_i_id, item_j_id, user_table, item_table,
             pred_i_out, pred_j_out,
             idx_u, idx_i, idx_j, rows_u, rows_i, rows_j,
             out_i_v, out_j_v, sem):
    wid = lax.axis_index("s") * _NC + lax.axis_index("c")
    base = wid * _BPW

    pltpu.sync_copy(user_id.at[pl.ds(base, _BPW)], idx_u)
    pltpu.sync_copy(item_i_id.at[pl.ds(base, _BPW)], idx_i)
    pltpu.sync_copy(item_j_id.at[pl.ds(base, _BPW)], idx_j)

    lanes = lax.iota(jnp.int32, _L)

    def _rot(x, d):
        return x.at[lanes ^ d].get(mode="promise_in_bounds", unique_indices=True)

    def _reduce16(ps):
        vs = ps
        d = _L // 2
        while len(vs) > 1:
            nxt = []
            for m in range(len(vs) // 2):
                a, b = vs[2 * m], vs[2 * m + 1]
                a2 = a + _rot(a, d)
                b2 = b + _rot(b, d)
                nxt.append(jnp.where((lanes & d) == 0, a2, b2))
            vs = nxt
            d //= 2
        br = (((lanes & 1) << 3) | ((lanes & 2) << 1)
              | ((lanes & 4) >> 1) | ((lanes & 8) >> 3))
        return vs[0].at[br].get(mode="promise_in_bounds", unique_indices=True)

    for chunk in range(_NCHUNK):
        c0 = chunk * _CH

        def issue_body(q, carry):
            rbase = c0 + q * _L
            vu = idx_u[pl.ds(rbase, _L)]
            vi = idx_i[pl.ds(rbase, _L)]
            vj = idx_j[pl.ds(rbase, _L)]
            for k in range(_L):
                dst = pl.ds(q * _L + k, 1)
                pltpu.async_copy(user_table.at[pl.ds(vu[k], 1), :], rows_u.at[dst, :], sem)
                pltpu.async_copy(item_table.at[pl.ds(vi[k], 1), :], rows_i.at[dst, :], sem)
                pltpu.async_copy(item_table.at[pl.ds(vj[k], 1), :], rows_j.at[dst, :], sem)
            return carry

        lax.fori_loop(0, _CH // _L, issue_body, 0)

        def wait_body(k, carry):
            pltpu.make_async_copy(user_table.at[pl.ds(0, 1), :], rows_u.at[pl.ds(k, 1), :], sem).wait()
            pltpu.make_async_copy(item_table.at[pl.ds(0, 1), :], rows_i.at[pl.ds(k, 1), :], sem).wait()
            pltpu.make_async_copy(item_table.at[pl.ds(0, 1), :], rows_j.at[pl.ds(k, 1), :], sem).wait()
            return carry

        lax.fori_loop(0, _CH, wait_body, 0)

        def group_body(g, carry):
            ps_i, ps_j = [], []
            for k in range(_L):
                r = g * _L + k
                p_i = jnp.zeros((_L,), jnp.float32)
                p_j = jnp.zeros((_L,), jnp.float32)
                for c in range(_D // _L):
                    sl = pl.ds(c * _L, _L)
                    u = rows_u[r, sl]
                    p_i = p_i + u * rows_i[r, sl]
                    p_j = p_j + u * rows_j[r, sl]
                ps_i.append(p_i)
                ps_j.append(p_j)
            out_i_v[pl.ds(c0 + g * _L, _L)] = _reduce16(ps_i)
            out_j_v[pl.ds(c0 + g * _L, _L)] = _reduce16(ps_j)
            return carry

        lax.fori_loop(0, _CH // _L, group_body, 0)

    pltpu.sync_copy(out_i_v, pred_i_out.at[pl.ds(base, _BPW)])
    pltpu.sync_copy(out_j_v, pred_j_out.at[pl.ds(base, _BPW)])


def kernel(user_id, item_i_id, item_j_id, user_table, item_table):
    f = pl.kernel(
        _sc_body,
        mesh=plsc.VectorSubcoreMesh(core_axis_name="c", subcore_axis_name="s"),
        compiler_params=pltpu.CompilerParams(use_tc_tiling_on_sc=True),
        out_type=(
            jax.ShapeDtypeStruct((_B,), jnp.float32),
            jax.ShapeDtypeStruct((_B,), jnp.float32),
        ),
        scratch_types=[
            pltpu.VMEM((_BPW,), jnp.int32),
            pltpu.VMEM((_BPW,), jnp.int32),
            pltpu.VMEM((_BPW,), jnp.int32),
            pltpu.VMEM((_CH, _D), jnp.float32),
            pltpu.VMEM((_CH, _D), jnp.float32),
            pltpu.VMEM((_CH, _D), jnp.float32),
            pltpu.VMEM((_BPW,), jnp.float32),
            pltpu.VMEM((_BPW,), jnp.float32),
            pltpu.SemaphoreType.DMA,
        ],
    )
    return f(user_id, item_i_id, item_j_id, user_table, item_table)
